# Initial kernel scaffold; baseline (speedup 1.0000x reference)
import functools

import jax
import jax.numpy as jnp
from jax import lax
from jax.experimental import pallas as pl
from jax.experimental.pallas import tpu as pltpu
from jax.experimental.pallas import tpu_sc as plsc

N = 10000
E = 320000
D = 128
NUM_ATOM = 100
L_LAYERS = 4
K_HOPS = 3

NC = 2
NS = 16
NW = NC * NS
EW = E // NW
CHUNK = 80
NCH = EW // CHUNK
RPS = N // NW
DEGW = 16

_mesh = plsc.VectorSubcoreMesh(
    core_axis_name="c", subcore_axis_name="s", num_cores=NC, num_subcores=NS
)


@functools.partial(
    pl.kernel,
    out_type=jax.ShapeDtypeStruct((NC, N, DEGW), jnp.float32),
    mesh=_mesh,
    scratch_types=[
        pltpu.VMEM((CHUNK,), jnp.int32),
        pltpu.VMEM((CHUNK, DEGW), jnp.float32),
        pltpu.VMEM_SHARED((N, DEGW), jnp.float32),
    ],
)
def _sc_degree(dst_hbm, ones_hbm, zeros_hbm, out_hbm, idx_d, ones_v, acc):
    c = lax.axis_index("c")
    s = lax.axis_index("s")
    w = c * NS + s
    pltpu.sync_copy(ones_hbm, ones_v)
    pltpu.sync_copy(zeros_hbm, acc.at[pl.ds(s * RPS, RPS)])
    plsc.subcore_barrier()

    def body(i, carry):
        pltpu.sync_copy(dst_hbm.at[w, i], idx_d)
        pltpu.sync_copy(ones_v, acc.at[idx_d], add=True)
        return carry

    lax.fori_loop(0, NCH, body, 0)
    plsc.subcore_barrier()
    pltpu.sync_copy(acc.at[pl.ds(s * RPS, RPS)], out_hbm.at[c, pl.ds(s * RPS, RPS)])


@functools.partial(
    pl.kernel,
    out_type=jax.ShapeDtypeStruct((NC, N, D), jnp.float32),
    mesh=_mesh,
    scratch_types=[
        pltpu.VMEM((CHUNK,), jnp.int32),
        pltpu.VMEM((CHUNK,), jnp.int32),
        pltpu.VMEM((CHUNK, D), jnp.float32),
        pltpu.VMEM_SHARED((N, D), jnp.float32),
        pltpu.SemaphoreType.DMA,
    ],
)
def _sc_round(xhat_hbm, src_hbm, dst_hbm, zeros_hbm, out_hbm,
              idx_s, idx_d, rows, acc, sem):
    c = lax.axis_index("c")
    s = lax.axis_index("s")
    w = c * NS + s
    pltpu.sync_copy(zeros_hbm, acc.at[pl.ds(s * RPS, RPS)])
    plsc.subcore_barrier()

    def body(i, carry):
        pltpu.sync_copy(src_hbm.at[w, i], idx_s)
        pltpu.sync_copy(dst_hbm.at[w, i], idx_d)
        pltpu.async_copy(xhat_hbm.at[idx_s], rows, sem).wait()
        pltpu.sync_copy(rows, acc.at[idx_d], add=True)
        return carry

    lax.fori_loop(0, NCH, body, 0)
    plsc.subcore_barrier()
    pltpu.sync_copy(acc.at[pl.ds(s * RPS, RPS)], out_hbm.at[c, pl.ds(s * RPS, RPS)])


def _tc_init_body(h_ref, emb_ref, degp_ref, w0_ref, gates_ref,
                  x_ref, isr_ref, xhat_ref, oacc_ref):
    h = h_ref[...]
    atoms = lax.broadcasted_iota(jnp.int32, (1, NUM_ATOM), 1)
    oh = (h == atoms).astype(jnp.float32)
    x = jnp.dot(oh, emb_ref[...], preferred_element_type=jnp.float32)
    deg = degp_ref[0, :, 0:1] + degp_ref[1, :, 0:1]
    isr = lax.rsqrt(jnp.maximum(deg, 1.0))
    isr_b = jnp.broadcast_to(isr, (N, D))
    g = jax.nn.sigmoid(gates_ref[0, 0])
    x_ref[...] = x
    isr_ref[...] = isr_b
    xhat_ref[...] = x * isr_b
    oacc_ref[...] = g * jnp.dot(x, w0_ref[...], preferred_element_type=jnp.float32)


def _tc_hop_body(l, k, parts_ref, isr_ref, w_ref, gates_ref, oacc_ref,
                 oacc_out_ref, xhat_out_ref):
    isr = isr_ref[...]
    xs = isr * (parts_ref[0] + parts_ref[1])
    g = jax.nn.sigmoid(gates_ref[l, k])
    oacc_out_ref[...] = oacc_ref[...] + g * jnp.dot(
        xs, w_ref[...], preferred_element_type=jnp.float32)
    xhat_out_ref[...] = xs * isr


def _layer_tail(l, parts_ref, isr_ref, w_ref, gates_ref, oacc_ref, hin_ref,
                snorm_ref, bns_ref, bnb_ref):
    isr = isr_ref[...]
    xs = isr * (parts_ref[0] + parts_ref[1])
    g = jax.nn.sigmoid(gates_ref[l, K_HOPS])
    out = oacc_ref[...] + g * jnp.dot(xs, w_ref[...],
                                      preferred_element_type=jnp.float32)
    out = out * snorm_ref[...]
    mu = jnp.mean(out, axis=0, keepdims=True)
    var = jnp.mean((out - mu) * (out - mu), axis=0, keepdims=True)
    out = (out - mu) / jnp.sqrt(var + 1e-5) * bns_ref[...] + bnb_ref[...]
    out = jnp.maximum(out, 0.0)
    return hin_ref[...] + out, isr


def _tc_tail_body(l, parts_ref, isr_ref, w_ref, gates_ref, oacc_ref, hin_ref,
                  snorm_ref, bns_ref, bnb_ref, wnext_ref,
                  xnew_ref, oaccn_ref, xhatn_ref):
    x_new, isr = _layer_tail(l, parts_ref, isr_ref, w_ref, gates_ref, oacc_ref,
                             hin_ref, snorm_ref, bns_ref, bnb_ref)
    gn = jax.nn.sigmoid(gates_ref[l + 1, 0])
    xnew_ref[...] = x_new
    oaccn_ref[...] = gn * jnp.dot(x_new, wnext_ref[...],
                                  preferred_element_type=jnp.float32)
    xhatn_ref[...] = x_new * isr


def _tc_final_body(l, parts_ref, isr_ref, w_ref, gates_ref, oacc_ref, hin_ref,
                   snorm_ref, bns_ref, bnb_ref,
                   w1_ref, b1_ref, w2_ref, b2_ref, w3_ref, b3_ref, y_ref):
    x_new, _ = _layer_tail(l, parts_ref, isr_ref, w_ref, gates_ref, oacc_ref,
                           hin_ref, snorm_ref, bns_ref, bnb_ref)
    hg = jnp.mean(x_new, axis=0, keepdims=True)
    y = jnp.dot(hg, w1_ref[...], preferred_element_type=jnp.float32) + b1_ref[...]
    y = jnp.maximum(y, 0.0)
    y = jnp.dot(y, w2_ref[...], preferred_element_type=jnp.float32) + b2_ref[...]
    y = jnp.maximum(y, 0.0)
    y_ref[...] = jnp.dot(y, w3_ref[...], preferred_element_type=jnp.float32) + b3_ref[...]


def _f32(shape):
    return jax.ShapeDtypeStruct(shape, jnp.float32)


def kernel(h, edge_index, e, snorm_n, snorm_e, emb, W, gates, bn_scale,
           bn_bias, w1, b1, w2, b2, w3, b3):
    src = edge_index[0].astype(jnp.int32).reshape(NW, NCH, CHUNK)
    dst = edge_index[1].astype(jnp.int32).reshape(NW, NCH, CHUNK)
    ones16 = jnp.ones((CHUNK, DEGW), jnp.float32)
    zeros16 = jnp.zeros((RPS, DEGW), jnp.float32)
    zerosD = jnp.zeros((RPS, D), jnp.float32)
    h2 = h.astype(jnp.int32).reshape(N, 1)

    degp = _sc_degree(dst, ones16, zeros16)

    x, isr, xhat, oacc = pl.pallas_call(
        _tc_init_body,
        out_shape=[_f32((N, D))] * 4,
    )(h2, emb, degp, W[0, 0], gates)

    y = None
    for l in range(L_LAYERS):
        for k in range(1, K_HOPS + 1):
            parts = _sc_round(xhat, src, dst, zerosD)
            if k < K_HOPS:
                oacc, xhat = pl.pallas_call(
                    functools.partial(_tc_hop_body, l, k),
                    out_shape=[_f32((N, D))] * 2,
                )(parts, isr, W[l, k], gates, oacc)
            elif l < L_LAYERS - 1:
                x, oacc, xhat = pl.pallas_call(
                    functools.partial(_tc_tail_body, l),
                    out_shape=[_f32((N, D))] * 3,
                )(parts, isr, W[l, K_HOPS], gates, oacc, x, snorm_n,
                  bn_scale[l].reshape(1, D), bn_bias[l].reshape(1, D),
                  W[l + 1, 0])
            else:
                y = pl.pallas_call(
                    functools.partial(_tc_final_body, l),
                    out_shape=_f32((1, 1)),
                )(parts, isr, W[l, K_HOPS], gates, oacc, x, snorm_n,
                  bn_scale[l].reshape(1, D), bn_bias[l].reshape(1, D),
                  w1, b1.reshape(1, D // 2), w2, b2.reshape(1, D // 4),
                  w3, b3.reshape(1, 1))
    return y


# speedup vs baseline: 5.2339x; 5.2339x over previous
import functools

import jax
import jax.numpy as jnp
from jax import lax
from jax.experimental import pallas as pl
from jax.experimental.pallas import tpu as pltpu
from jax.experimental.pallas import tpu_sc as plsc

N = 10000
E = 320000
D = 128
NUM_ATOM = 100
L_LAYERS = 4
K_HOPS = 3

NC = 2
NS = 16
NW = NC * NS
EW = E // NW
CHUNK = 80
NCH = EW // CHUNK
NP = 10240
RPS = NP // NS


@functools.cache
def _sc_kernels():
    mesh = plsc.VectorSubcoreMesh(
        core_axis_name="c", subcore_axis_name="s",
        num_cores=NC, num_subcores=NS,
    )
    sc_degree = pl.kernel(
        _sc_degree_body,
        out_type=jax.ShapeDtypeStruct((NC, NP, D), jnp.float32),
        mesh=mesh,
        scratch_types=[
            pltpu.VMEM((CHUNK,), jnp.int32),
            pltpu.VMEM((CHUNK, D), jnp.float32),
            pltpu.VMEM_SHARED((NP, D), jnp.float32),
        ],
    )
    sc_round = pl.kernel(
        _sc_round_body,
        out_type=jax.ShapeDtypeStruct((NC, NP, D), jnp.float32),
        mesh=mesh,
        scratch_types=[
            pltpu.VMEM((CHUNK,), jnp.int32),
            pltpu.VMEM((CHUNK,), jnp.int32),
            pltpu.VMEM((CHUNK, D), jnp.float32),
            pltpu.VMEM_SHARED((NP, D), jnp.float32),
            pltpu.SemaphoreType.DMA,
        ],
    )
    return sc_degree, sc_round


def _sc_degree_body(dst_hbm, ones_hbm, zeros_hbm, out_hbm, idx_d, ones_v, acc):
    c = lax.axis_index("c")
    s = lax.axis_index("s")
    w = c * NS + s
    pltpu.sync_copy(ones_hbm, ones_v)
    pltpu.sync_copy(zeros_hbm, acc.at[pl.ds(s * RPS, RPS)])
    plsc.subcore_barrier()

    def body(i, carry):
        pltpu.sync_copy(dst_hbm.at[w, i], idx_d)
        pltpu.sync_copy(ones_v, acc.at[idx_d], add=True)
        return carry

    lax.fori_loop(0, NCH, body, 0)
    plsc.subcore_barrier()
    pltpu.sync_copy(acc.at[pl.ds(s * RPS, RPS)], out_hbm.at[c, pl.ds(s * RPS, RPS)])


def _sc_round_body(xhat_hbm, src_hbm, dst_hbm, zeros_hbm, out_hbm,
                   idx_s, idx_d, rows, acc, sem):
    c = lax.axis_index("c")
    s = lax.axis_index("s")
    w = c * NS + s
    pltpu.sync_copy(zeros_hbm, acc.at[pl.ds(s * RPS, RPS)])
    plsc.subcore_barrier()

    def body(i, carry):
        pltpu.sync_copy(src_hbm.at[w, i], idx_s)
        pltpu.sync_copy(dst_hbm.at[w, i], idx_d)
        pltpu.async_copy(xhat_hbm.at[idx_s], rows, sem).wait()
        pltpu.sync_copy(rows, acc.at[idx_d], add=True)
        return carry

    lax.fori_loop(0, NCH, body, 0)
    plsc.subcore_barrier()
    pltpu.sync_copy(acc.at[pl.ds(s * RPS, RPS)], out_hbm.at[c, pl.ds(s * RPS, RPS)])


def _tc_init_body(h_ref, emb_ref, degp_ref, w0_ref, gates_ref,
                  x_ref, isr_ref, xhat_ref, oacc_ref):
    h = h_ref[...]
    atoms = lax.broadcasted_iota(jnp.int32, (1, NUM_ATOM), 1)
    oh = (h == atoms).astype(jnp.float32)
    x = jnp.dot(oh, emb_ref[...], preferred_element_type=jnp.float32)
    deg = degp_ref[0, :N, 0:1] + degp_ref[1, :N, 0:1]
    isr = lax.rsqrt(jnp.maximum(deg, 1.0))
    isr_b = jnp.broadcast_to(isr, (N, D))
    g = jax.nn.sigmoid(gates_ref[0, 0])
    x_ref[...] = x
    isr_ref[...] = isr_b
    xhat_ref[...] = x * isr_b
    oacc_ref[...] = g * jnp.dot(x, w0_ref[...], preferred_element_type=jnp.float32)


def _tc_hop_body(l, k, parts_ref, isr_ref, w_ref, gates_ref, oacc_ref,
                 oacc_out_ref, xhat_out_ref):
    isr = isr_ref[...]
    xs = isr * (parts_ref[0, :N] + parts_ref[1, :N])
    g = jax.nn.sigmoid(gates_ref[l, k])
    oacc_out_ref[...] = oacc_ref[...] + g * jnp.dot(
        xs, w_ref[...], preferred_element_type=jnp.float32)
    xhat_out_ref[...] = xs * isr


def _layer_tail(l, parts_ref, isr_ref, w_ref, gates_ref, oacc_ref, hin_ref,
                snorm_ref, bns_ref, bnb_ref):
    isr = isr_ref[...]
    xs = isr * (parts_ref[0, :N] + parts_ref[1, :N])
    g = jax.nn.sigmoid(gates_ref[l, K_HOPS])
    out = oacc_ref[...] + g * jnp.dot(xs, w_ref[...],
                                      preferred_element_type=jnp.float32)
    out = out * snorm_ref[...]
    mu = jnp.mean(out, axis=0, keepdims=True)
    var = jnp.mean((out - mu) * (out - mu), axis=0, keepdims=True)
    out = (out - mu) / jnp.sqrt(var + 1e-5) * bns_ref[...] + bnb_ref[...]
    out = jnp.maximum(out, 0.0)
    return hin_ref[...] + out, isr


def _tc_tail_body(l, parts_ref, isr_ref, w_ref, gates_ref, oacc_ref, hin_ref,
                  snorm_ref, bns_ref, bnb_ref, wnext_ref,
                  xnew_ref, oaccn_ref, xhatn_ref):
    x_new, isr = _layer_tail(l, parts_ref, isr_ref, w_ref, gates_ref, oacc_ref,
                             hin_ref, snorm_ref, bns_ref, bnb_ref)
    gn = jax.nn.sigmoid(gates_ref[l + 1, 0])
    xnew_ref[...] = x_new
    oaccn_ref[...] = gn * jnp.dot(x_new, wnext_ref[...],
                                  preferred_element_type=jnp.float32)
    xhatn_ref[...] = x_new * isr


def _tc_final_body(l, parts_ref, isr_ref, w_ref, gates_ref, oacc_ref, hin_ref,
                   snorm_ref, bns_ref, bnb_ref,
                   w1_ref, b1_ref, w2_ref, b2_ref, w3_ref, b3_ref, y_ref):
    x_new, _ = _layer_tail(l, parts_ref, isr_ref, w_ref, gates_ref, oacc_ref,
                           hin_ref, snorm_ref, bns_ref, bnb_ref)
    hg = jnp.mean(x_new, axis=0, keepdims=True)
    y = jnp.dot(hg, w1_ref[...], preferred_element_type=jnp.float32) + b1_ref[...]
    y = jnp.maximum(y, 0.0)
    y = jnp.dot(y, w2_ref[...], preferred_element_type=jnp.float32) + b2_ref[...]
    y = jnp.maximum(y, 0.0)
    y_ref[...] = jnp.dot(y, w3_ref[...], preferred_element_type=jnp.float32) + b3_ref[...]


def _f32(shape):
    return jax.ShapeDtypeStruct(shape, jnp.float32)


def kernel(h, edge_index, e, snorm_n, snorm_e, emb, W, gates, bn_scale,
           bn_bias, w1, b1, w2, b2, w3, b3):
    src = edge_index[0].astype(jnp.int32).reshape(NW, NCH, CHUNK)
    dst = edge_index[1].astype(jnp.int32).reshape(NW, NCH, CHUNK)
    onesD = jnp.ones((CHUNK, D), jnp.float32)
    zerosD = jnp.zeros((RPS, D), jnp.float32)
    h2 = h.astype(jnp.int32).reshape(N, 1)

    sc_degree, sc_round = _sc_kernels()
    degp = sc_degree(dst, onesD, zerosD)

    x, isr, xhat, oacc = pl.pallas_call(
        _tc_init_body,
        out_shape=[_f32((N, D))] * 4,
    )(h2, emb, degp, W[0, 0], gates)

    y = None
    for l in range(L_LAYERS):
        for k in range(1, K_HOPS + 1):
            parts = sc_round(xhat, src, dst, zerosD)
            if k < K_HOPS:
                oacc, xhat = pl.pallas_call(
                    functools.partial(_tc_hop_body, l, k),
                    out_shape=[_f32((N, D))] * 2,
                )(parts, isr, W[l, k], gates, oacc)
            elif l < L_LAYERS - 1:
                x, oacc, xhat = pl.pallas_call(
                    functools.partial(_tc_tail_body, l),
                    out_shape=[_f32((N, D))] * 3,
                )(parts, isr, W[l, K_HOPS], gates, oacc, x, snorm_n,
                  bn_scale[l].reshape(1, D), bn_bias[l].reshape(1, D),
                  W[l + 1, 0])
            else:
                y = pl.pallas_call(
                    functools.partial(_tc_final_body, l),
                    out_shape=_f32((1, 1)),
                )(parts, isr, W[l, K_HOPS], gates, oacc, x, snorm_n,
                  bn_scale[l].reshape(1, D), bn_bias[l].reshape(1, D),
                  w1, b1.reshape(1, D // 2), w2, b2.reshape(1, D // 4),
                  w3, b3.reshape(1, 1))
    return y
